# baseline (speedup 1.0000x reference)
import functools

import jax
import jax.numpy as jnp
from jax import lax
from jax.experimental import pallas as pl
from jax.experimental.pallas import tpu as pltpu
from jax.experimental.pallas import tpu_sc as plsc

EMBED_DIM = 64
BS = 4096
NUM_AXIOMS = 200
N_PAIRS = BS * NUM_AXIOMS
N_ENT = 2 * N_PAIRS
NW = 32
ENT_PER_W = N_ENT // NW
PAIRS_PER_W = N_PAIRS // NW
CHUNK_ENT = 512
CHUNK_PAIRS = CHUNK_ENT // 2
N_CHUNKS = ENT_PER_W // CHUNK_ENT
IDX_MINOR = 128
SUB = CHUNK_ENT // IDX_MINOR
IDX_ROWS_PER_W = ENT_PER_W // IDX_MINOR


def _sc_score(x2d, emb):
    mesh = plsc.VectorSubcoreMesh(core_axis_name="c", subcore_axis_name="s")

    @functools.partial(
        pl.kernel,
        mesh=mesh,
        out_type=jax.ShapeDtypeStruct((N_PAIRS,), jnp.float32),
        compiler_params=pltpu.CompilerParams(
            needs_layout_passes=False, use_tc_tiling_on_sc=False),
        scratch_types=[
            pltpu.VMEM((IDX_ROWS_PER_W, IDX_MINOR), jnp.int32),
            pltpu.VMEM((CHUNK_ENT, EMBED_DIM), jnp.float32),
            pltpu.VMEM((CHUNK_ENT, EMBED_DIM), jnp.float32),
            pltpu.VMEM((CHUNK_PAIRS,), jnp.float32),
            pltpu.SemaphoreType.DMA,
            pltpu.SemaphoreType.DMA,
        ],
    )
    def k(x_hbm, emb_hbm, out_hbm, idx_v, rows0, rows1, out_v, sem0, sem1):
        wid = lax.axis_index("s") * 2 + lax.axis_index("c")
        pltpu.sync_copy(
            x_hbm.at[pl.ds(wid * IDX_ROWS_PER_W, IDX_ROWS_PER_W)], idx_v)

        def fire(g, rows, sem):
            for j in range(SUB):
                pltpu.async_copy(
                    emb_hbm.at[idx_v.at[g * SUB + j]],
                    rows.at[pl.ds(j * IDX_MINOR, IDX_MINOR)],
                    sem)

        def drain(rows, sem):
            pltpu.make_async_copy(
                emb_hbm.at[pl.ds(0, CHUNK_ENT)], rows, sem).wait()

        lanes = lax.iota(jnp.int32, 16)

        def compute(g, rows):
            def group(t, carry):
                row_l = 32 * t + 2 * lanes
                row_r = row_l + 1
                acc = jnp.zeros((16,), jnp.float32)
                for d in range(EMBED_DIM):
                    col = jnp.full((16,), d, jnp.int32)
                    vl = plsc.load_gather(rows, [row_l, col])
                    vr = plsc.load_gather(rows, [row_r, col])
                    acc = acc + vl * vr
                out_v[pl.ds(t * 16, 16)] = acc
                return carry
            lax.fori_loop(0, CHUNK_PAIRS // 16, group, 0)
            out_base = wid * PAIRS_PER_W + g * CHUNK_PAIRS
            pltpu.sync_copy(out_v, out_hbm.at[pl.ds(out_base, CHUNK_PAIRS)])

        fire(0, rows0, sem0)

        def body(i, carry):
            g0 = 2 * i
            fire(g0 + 1, rows1, sem1)
            drain(rows0, sem0)
            compute(g0, rows0)

            @pl.when(i < N_CHUNKS // 2 - 1)
            def _():
                fire(g0 + 2, rows0, sem0)

            drain(rows1, sem1)
            compute(g0 + 1, rows1)
            return carry

        lax.fori_loop(0, N_CHUNKS // 2, body, 0)

    return k(x2d, emb)


def kernel(x, emb):
    x2d = x.astype(jnp.int32).reshape(N_ENT // IDX_MINOR, IDX_MINOR)
    scores = _sc_score(x2d, emb.astype(jnp.float32))
    return scores.reshape(BS, NUM_AXIOMS)


# speedup vs baseline: 1.7215x; 1.7215x over previous
import functools

import jax
import jax.numpy as jnp
from jax import lax
from jax.experimental import pallas as pl
from jax.experimental.pallas import tpu as pltpu
from jax.experimental.pallas import tpu_sc as plsc

EMBED_DIM = 64
BS = 4096
NUM_AXIOMS = 200
N_PAIRS = BS * NUM_AXIOMS
N_ENT = 2 * N_PAIRS
NW = 32
ENT_PER_W = N_ENT // NW
PAIRS_PER_W = N_PAIRS // NW
CHUNK_ENT = 512
CHUNK_PAIRS = CHUNK_ENT // 2
N_CHUNKS = ENT_PER_W // CHUNK_ENT
IDX_MINOR = 128
SUB = CHUNK_ENT // IDX_MINOR
IDX_ROWS_PER_W = ENT_PER_W // IDX_MINOR


def _sc_score(x2d, emb):
    mesh = plsc.VectorSubcoreMesh(core_axis_name="c", subcore_axis_name="s")

    @functools.partial(
        pl.kernel,
        mesh=mesh,
        out_type=jax.ShapeDtypeStruct((N_PAIRS,), jnp.float32),
        compiler_params=pltpu.CompilerParams(
            needs_layout_passes=False, use_tc_tiling_on_sc=False),
        scratch_types=[
            pltpu.VMEM((IDX_ROWS_PER_W, IDX_MINOR), jnp.int32),
            pltpu.VMEM((CHUNK_ENT, EMBED_DIM), jnp.float32),
            pltpu.VMEM((CHUNK_ENT, EMBED_DIM), jnp.float32),
            pltpu.VMEM((CHUNK_PAIRS,), jnp.float32),
            pltpu.SemaphoreType.DMA,
            pltpu.SemaphoreType.DMA,
        ],
    )
    def k(x_hbm, emb_hbm, out_hbm, idx_v, rows0, rows1, out_v, sem0, sem1):
        wid = lax.axis_index("s") * 2 + lax.axis_index("c")
        pltpu.sync_copy(
            x_hbm.at[pl.ds(wid * IDX_ROWS_PER_W, IDX_ROWS_PER_W)], idx_v)

        def fire(g, rows, sem):
            for j in range(SUB):
                pltpu.async_copy(
                    emb_hbm.at[idx_v.at[g * SUB + j]],
                    rows.at[pl.ds(j * IDX_MINOR, IDX_MINOR)],
                    sem)

        def drain(rows, sem):
            pltpu.make_async_copy(
                emb_hbm.at[pl.ds(0, CHUNK_ENT)], rows, sem).wait()

        lanes = lax.iota(jnp.int32, 16)

        def compute(g, rows):
            def group(t, carry):
                row_l = 32 * t + 2 * lanes
                row_r = row_l + 1
                base_col = lanes + t
                acc = jnp.zeros((16,), jnp.float32)
                for d in range(EMBED_DIM):
                    col = lax.bitwise_and(base_col + d, 63)
                    vl = plsc.load_gather(rows, [row_l, col])
                    vr = plsc.load_gather(rows, [row_r, col])
                    acc = acc + vl * vr
                out_v[pl.ds(t * 16, 16)] = acc
                return carry
            lax.fori_loop(0, CHUNK_PAIRS // 16, group, 0)
            out_base = wid * PAIRS_PER_W + g * CHUNK_PAIRS
            pltpu.sync_copy(out_v, out_hbm.at[pl.ds(out_base, CHUNK_PAIRS)])

        fire(0, rows0, sem0)

        def body(i, carry):
            g0 = 2 * i
            fire(g0 + 1, rows1, sem1)
            drain(rows0, sem0)
            compute(g0, rows0)

            @pl.when(i < N_CHUNKS // 2 - 1)
            def _():
                fire(g0 + 2, rows0, sem0)

            drain(rows1, sem1)
            compute(g0 + 1, rows1)
            return carry

        lax.fori_loop(0, N_CHUNKS // 2, body, 0)

    return k(x2d, emb)


def kernel(x, emb):
    x2d = x.astype(jnp.int32).reshape(N_ENT // IDX_MINOR, IDX_MINOR)
    scores = _sc_score(x2d, emb.astype(jnp.float32))
    return scores.reshape(BS, NUM_AXIOMS)
